# baseline (speedup 1.0000x reference)
import functools

import jax
import jax.numpy as jnp
from jax import lax
from jax.experimental import pallas as pl
from jax.experimental.pallas import tpu as pltpu
from jax.experimental.pallas import tpu_sc as plsc

BATCH = 4096
HIST = 50
HISTP = 64
HISTG = 56
EMBED = 64
VOCAB = 1000000
NC = 2
NS = 16
NW = NC * NS
BATW = BATCH // NW
NB = 2
ROUNDS = BATW // NB


def _emb_body(idx_hbm, table_hbm, out_hbm, idx_v, gidx_v, off_v, rows_v,
              out_v, sem_g, sem_s):
    wid = lax.axis_index("s") * NC + lax.axis_index("c")
    bbase = wid * BATW
    pltpu.sync_copy(idx_hbm.at[pl.ds(bbase, BATW)], idx_v)

    def prep_and_fire(rr, b):
        for bi in range(NB):
            for c in range(HISTP // 16):
                v = idx_v[rr * NB + bi, pl.ds(c * 16, 16)]
                gidx_v[b, bi, pl.ds(c * 16, 16)] = v >> 1
                off_v[b, bi, pl.ds(c * 16, 16)] = v & 1
            pltpu.async_copy(
                table_hbm.at[gidx_v.at[b, bi, pl.ds(0, HISTG)]],
                rows_v.at[b, bi],
                sem_g.at[b],
            )

    def drain_gathers(b):
        for bi in range(NB):
            pltpu.make_async_copy(
                table_hbm.at[gidx_v.at[b, bi, pl.ds(0, HISTG)]],
                rows_v.at[b, bi],
                sem_g.at[b],
            ).wait()

    def wait_store(b):
        pltpu.make_async_copy(
            out_v.at[b],
            out_hbm.at[pl.ds(0, NB)],
            sem_s.at[b],
        ).wait()

    prep_and_fire(0, 0)

    def round_step(r, buf):
        other = 1 - buf
        drain_gathers(buf)

        @pl.when(r >= 2)
        def _():
            wait_store(buf)

        @pl.when(r + 1 < ROUNDS)
        def _():
            prep_and_fire(r + 1, other)

        for bi in range(NB):
            def compact(jj, _, bi=bi):
                j0 = jj * 2
                blk = (jj >> 3) * 16
                goff = off_v[buf, bi, pl.ds(blk, 16)]
                for t in range(2):
                    j = j0 + t
                    spl = goff.at[jnp.full((16,), j - blk, jnp.int32)].get(
                        mode="promise_in_bounds")
                    f = spl.astype(jnp.float32)
                    for k in range(EMBED // 16):
                        left = rows_v[buf, bi, j, pl.ds(k * 16, 16)]
                        right = rows_v[buf, bi, j, pl.ds(64 + k * 16, 16)]
                        out_v[buf, bi, j, pl.ds(k * 16, 16)] = (
                            left + (right - left) * f
                        )
                return 0

            lax.fori_loop(0, HIST // 2, compact, 0)

        pltpu.async_copy(
            out_v.at[buf],
            out_hbm.at[pl.ds(bbase + r * NB, NB)],
            sem_s.at[buf],
        )

    def body(i, _):
        round_step(2 * i, 0)
        round_step(2 * i + 1, 1)
        return 0

    lax.fori_loop(0, ROUNDS // 2, body, 0)

    wait_store(0)
    wait_store(1)


@jax.jit
def _emb(idxp, table2):
    mesh = plsc.VectorSubcoreMesh(core_axis_name="c", subcore_axis_name="s")
    f = functools.partial(
        pl.kernel,
        mesh=mesh,
        out_type=jax.ShapeDtypeStruct((BATCH, HIST, EMBED), jnp.float32),
        scratch_types=[
            pltpu.VMEM((BATW, HISTP), jnp.int32),
            pltpu.VMEM((2, NB, HISTP), jnp.int32),
            pltpu.VMEM((2, NB, HISTP), jnp.int32),
            pltpu.VMEM((2, NB, HISTG, 128), jnp.float32),
            pltpu.VMEM((2, NB, HIST, EMBED), jnp.float32),
            pltpu.SemaphoreType.DMA((2,)),
            pltpu.SemaphoreType.DMA((2,)),
        ],
    )(_emb_body)
    return f(idxp, table2)


def kernel(indices, table):
    idxp = jnp.pad(indices, ((0, 0), (0, HISTP - HIST)))
    table2 = table.reshape(VOCAB // 2, 2 * EMBED)
    return _emb(idxp, table2)
